# Initial kernel scaffold; baseline (speedup 1.0000x reference)
import jax
import jax.numpy as jnp
from jax.experimental import pallas as pl


def kernel(reconstructed_image, target_image):
    raise NotImplementedError("write your pallas kernel here")


# speedup vs baseline: 47.2988x; 47.2988x over previous
import functools

import jax
import jax.numpy as jnp
from jax import lax
from jax.experimental import pallas as pl
from jax.experimental.pallas import tpu as pltpu
from jax.experimental.pallas import tpu_sc as plsc

_TIMESTEPS = 1000
_NROWS = 2048
_W = 256
_NWORKERS = 32
_ROWS_PER_W = _NROWS // _NWORKERS
_HOLO = 1024
_LANES = 16


def _build_holo_row(buf, r, hbuf, nbuf, lane_i32, lane_f32):
    for g in range(_W // _LANES):
        v = buf[r, pl.ds(g * _LANES, _LANES)]
        q0 = (v * 1000.0).astype(jnp.int32) - 1
        q = jnp.where(q0 < 0, q0 + _TIMESTEPS, q0)
        valid = v != 0.0
        key = jnp.where(valid, q * _LANES + lane_i32,
                        jnp.int32(0x7FFF0000) + lane_i32)
        val = jnp.where(valid, jnp.float32(g * _LANES) + lane_f32,
                        jnp.float32(-1.0))
        skey, sval = plsc.sort_key_val(key, val)
        nbuf[pl.ds(0, _LANES)] = skey
        nkey = nbuf[pl.ds(1, _LANES)]
        qs = lax.shift_right_logical(skey, 4)
        nq = lax.shift_right_logical(nkey, 4)
        winner = (qs != nq) & (sval >= 0.0)
        idx = jnp.minimum(qs, _HOLO - 1)
        plsc.store_scatter(hbuf, [idx], sval, mask=winner)


def _sc_loss_kernel(rec_hbm, tgt_hbm, out_hbm, rbuf, tbuf, hr, ht, nbuf,
                    accv, sem_r, sem_t):
    wid = lax.axis_index("c") * 16 + lax.axis_index("s")
    base = wid * _ROWS_PER_W

    cp_r = pltpu.make_async_copy(rec_hbm.at[pl.ds(base, _ROWS_PER_W)],
                                 rbuf, sem_r)
    cp_t = pltpu.make_async_copy(tgt_hbm.at[pl.ds(base, _ROWS_PER_W)],
                                 tbuf, sem_t)
    cp_r.start()
    cp_t.start()

    lane_i32 = lax.iota(jnp.int32, _LANES)
    lane_f32 = lane_i32.astype(jnp.float32)
    zf = jnp.zeros((_LANES,), jnp.float32)

    nbuf[pl.ds(_LANES, _LANES)] = jnp.full((_LANES,), -1, jnp.int32)
    for j in range(_HOLO // _LANES):
        hr[pl.ds(j * _LANES, _LANES)] = zf
        ht[pl.ds(j * _LANES, _LANES)] = zf

    cp_r.wait()
    cp_t.wait()

    def row_body(r, acc):
        _build_holo_row(rbuf, r, hr, nbuf, lane_i32, lane_f32)
        _build_holo_row(tbuf, r, ht, nbuf, lane_i32, lane_f32)
        for j in range(_HOLO // _LANES):
            sl = pl.ds(j * _LANES, _LANES)
            d = hr[sl] - ht[sl]
            acc = acc + d * d
            hr[sl] = zf
            ht[sl] = zf
        return acc

    acc = lax.fori_loop(0, _ROWS_PER_W, row_body, jnp.zeros((_LANES,),
                                                            jnp.float32))
    accv[...] = acc
    pltpu.sync_copy(accv, out_hbm.at[wid])


@jax.jit
def kernel(reconstructed_image, target_image):
    rec = jnp.reshape(reconstructed_image, (_NROWS, _W))
    tgt = jnp.reshape(target_image, (_NROWS, _W))

    mesh = plsc.VectorSubcoreMesh(core_axis_name="c", subcore_axis_name="s")
    partials = pl.kernel(
        _sc_loss_kernel,
        mesh=mesh,
        compiler_params=pltpu.CompilerParams(needs_layout_passes=False),
        out_type=jax.ShapeDtypeStruct((_NWORKERS, _LANES), jnp.float32),
        scratch_types=[
            pltpu.VMEM((_ROWS_PER_W, _W), jnp.float32),
            pltpu.VMEM((_ROWS_PER_W, _W), jnp.float32),
            pltpu.VMEM((_HOLO,), jnp.float32),
            pltpu.VMEM((_HOLO,), jnp.float32),
            pltpu.VMEM((2 * _LANES,), jnp.int32),
            pltpu.VMEM((_LANES,), jnp.float32),
            pltpu.SemaphoreType.DMA,
            pltpu.SemaphoreType.DMA,
        ],
    )(rec, tgt)

    denom = jnp.float32(8 * 1 * 256 * _TIMESTEPS)
    return jnp.sum(partials) / denom


# speedup vs baseline: 59.6004x; 1.2601x over previous
import functools

import jax
import jax.numpy as jnp
from jax import lax
from jax.experimental import pallas as pl
from jax.experimental.pallas import tpu as pltpu
from jax.experimental.pallas import tpu_sc as plsc

_TIMESTEPS = 1000
_NROWS = 2048
_W = 256
_NWORKERS = 32
_ROWS_PER_W = _NROWS // _NWORKERS
_HOLO = 1024
_LANES = 16


def _build_holo_row(buf, r, hbuf, lane_f32):
    for g in range(_W // _LANES):
        v = buf[r, pl.ds(g * _LANES, _LANES)]
        q0 = (v * 1000.0).astype(jnp.int32) - 1
        q = jnp.where(q0 < 0, _TIMESTEPS - 1, q0)
        valid = v != 0.0
        _, winner = plsc.scan_count(q, mask=valid)
        val = jnp.float32(g * _LANES) + lane_f32
        plsc.store_scatter(hbuf, [q], val, mask=winner)


def _sc_loss_kernel(rec_hbm, tgt_hbm, out_hbm, rbuf, tbuf, hr, ht,
                    accv, sem_r, sem_t):
    wid = lax.axis_index("c") * 16 + lax.axis_index("s")
    base = wid * _ROWS_PER_W

    cp_r = pltpu.make_async_copy(rec_hbm.at[pl.ds(base, _ROWS_PER_W)],
                                 rbuf, sem_r)
    cp_t = pltpu.make_async_copy(tgt_hbm.at[pl.ds(base, _ROWS_PER_W)],
                                 tbuf, sem_t)
    cp_r.start()
    cp_t.start()

    lane_i32 = lax.iota(jnp.int32, _LANES)
    lane_f32 = lane_i32.astype(jnp.float32)
    zf = jnp.zeros((_LANES,), jnp.float32)

    for j in range(_HOLO // _LANES):
        hr[pl.ds(j * _LANES, _LANES)] = zf
        ht[pl.ds(j * _LANES, _LANES)] = zf

    cp_r.wait()
    cp_t.wait()

    def row_body(r, acc):
        _build_holo_row(rbuf, r, hr, lane_f32)
        _build_holo_row(tbuf, r, ht, lane_f32)
        for j in range(_HOLO // _LANES):
            sl = pl.ds(j * _LANES, _LANES)
            d = hr[sl] - ht[sl]
            acc = acc + d * d
            hr[sl] = zf
            ht[sl] = zf
        return acc

    acc = lax.fori_loop(0, _ROWS_PER_W, row_body, jnp.zeros((_LANES,),
                                                            jnp.float32))
    accv[...] = acc
    pltpu.sync_copy(accv, out_hbm.at[wid])


@jax.jit
def kernel(reconstructed_image, target_image):
    rec = jnp.reshape(reconstructed_image, (_NROWS, _W))
    tgt = jnp.reshape(target_image, (_NROWS, _W))

    mesh = plsc.VectorSubcoreMesh(core_axis_name="c", subcore_axis_name="s")
    partials = pl.kernel(
        _sc_loss_kernel,
        mesh=mesh,
        compiler_params=pltpu.CompilerParams(needs_layout_passes=False),
        out_type=jax.ShapeDtypeStruct((_NWORKERS, _LANES), jnp.float32),
        scratch_types=[
            pltpu.VMEM((_ROWS_PER_W, _W), jnp.float32),
            pltpu.VMEM((_ROWS_PER_W, _W), jnp.float32),
            pltpu.VMEM((_HOLO,), jnp.float32),
            pltpu.VMEM((_HOLO,), jnp.float32),
            pltpu.VMEM((_LANES,), jnp.float32),
            pltpu.SemaphoreType.DMA,
            pltpu.SemaphoreType.DMA,
        ],
    )(rec, tgt)

    denom = jnp.float32(8 * 1 * 256 * _TIMESTEPS)
    return jnp.sum(partials) / denom
